# baseline (speedup 1.0000x reference)
import functools

import jax
import jax.numpy as jnp
from jax import lax
from jax.experimental import pallas as pl
from jax.experimental.pallas import tpu as pltpu
from jax.experimental.pallas import tpu_sc as plsc

D_MODEL = 1024
SEQ_LEN = 2048
BATCH = 4

_INFO = plsc.get_sparse_core_info()
_NC, _NS, _L = _INFO.num_cores, _INFO.num_subcores, _INFO.num_lanes
_NW = _NC * _NS

ROWS_PER_W = SEQ_LEN // _NW
PE_WORDS = ROWS_PER_W * D_MODEL
CHUNK_ROWS = 16
CHUNK_WORDS = CHUNK_ROWS * D_MODEL
CHUNKS_PER_BAND = ROWS_PER_W // CHUNK_ROWS
UNROLL = 8

_mesh = plsc.VectorSubcoreMesh(core_axis_name="c", subcore_axis_name="s")


@functools.partial(
    pl.kernel,
    mesh=_mesh,
    out_type=jax.ShapeDtypeStruct((BATCH * SEQ_LEN * D_MODEL,), jnp.float32),
    scratch_types=[
        pltpu.VMEM((PE_WORDS,), jnp.float32),
        pltpu.VMEM((CHUNK_WORDS,), jnp.float32),
        pltpu.SemaphoreType.DMA,
    ],
)
def _pe_add(x_hbm, pe_hbm, out_hbm, pe_v, buf, sem):
    wid = lax.axis_index("s") * _NC + lax.axis_index("c")
    pe_base = wid * PE_WORDS

    pltpu.sync_copy(pe_hbm.at[pl.ds(pe_base, PE_WORDS)], pe_v)

    for b in range(BATCH):
        for c in range(CHUNKS_PER_BAND):
            xoff = b * (SEQ_LEN * D_MODEL) + pe_base + c * CHUNK_WORDS
            pltpu.sync_copy(x_hbm.at[pl.ds(xoff, CHUNK_WORDS)], buf)

            pe_off = c * CHUNK_WORDS

            def body(i, _, pe_off=pe_off):
                base = i * (_L * UNROLL)
                for u in range(UNROLL):
                    o = base + u * _L
                    buf[pl.ds(o, _L)] = (
                        buf[pl.ds(o, _L)] + pe_v[pl.ds(pe_off + o, _L)]
                    )
                return 0

            lax.fori_loop(0, CHUNK_WORDS // (_L * UNROLL), body, 0)

            pltpu.sync_copy(buf, out_hbm.at[pl.ds(xoff, CHUNK_WORDS)])


def kernel(x, pe):
    out_flat = _pe_add(x.reshape(-1), pe.reshape(-1))
    return out_flat.reshape(x.shape)


# speedup vs baseline: 1.2064x; 1.2064x over previous
import functools

import jax
import jax.numpy as jnp
from jax import lax
from jax.experimental import pallas as pl
from jax.experimental.pallas import tpu as pltpu
from jax.experimental.pallas import tpu_sc as plsc

D_MODEL = 1024
SEQ_LEN = 2048
BATCH = 4

_INFO = plsc.get_sparse_core_info()
_NC, _NS, _L = _INFO.num_cores, _INFO.num_subcores, _INFO.num_lanes
_NW = _NC * _NS

ROWS_PER_W = SEQ_LEN // _NW
PE_WORDS = ROWS_PER_W * D_MODEL
CHUNK_ROWS = 16
CHUNK_WORDS = CHUNK_ROWS * D_MODEL
CHUNKS_PER_BAND = ROWS_PER_W // CHUNK_ROWS
N_CHUNKS = BATCH * CHUNKS_PER_BAND
NSLOTS = 3
UNROLL = 8

_mesh = plsc.VectorSubcoreMesh(core_axis_name="c", subcore_axis_name="s")


def _chunk_off(wid, k):
    b, c = divmod(k, CHUNKS_PER_BAND)
    return b * (SEQ_LEN * D_MODEL) + wid * PE_WORDS + c * CHUNK_WORDS


@functools.partial(
    pl.kernel,
    mesh=_mesh,
    out_type=jax.ShapeDtypeStruct((BATCH * SEQ_LEN * D_MODEL,), jnp.float32),
    scratch_types=[
        pltpu.VMEM((PE_WORDS,), jnp.float32),
    ]
    + [pltpu.VMEM((CHUNK_WORDS,), jnp.float32) for _ in range(NSLOTS)]
    + [pltpu.SemaphoreType.DMA for _ in range(2 * NSLOTS)],
)
def _pe_add(x_hbm, pe_hbm, out_hbm, pe_v, b0, b1, b2, i0, i1, i2, o0, o1, o2):
    bufs = (b0, b1, b2)
    in_sems = (i0, i1, i2)
    out_sems = (o0, o1, o2)

    wid = lax.axis_index("s") * _NC + lax.axis_index("c")
    pe_base = wid * PE_WORDS

    pltpu.sync_copy(pe_hbm.at[pl.ds(pe_base, PE_WORDS)], pe_v)

    def start_in(k):
        s = k % NSLOTS
        pltpu.async_copy(
            x_hbm.at[pl.ds(_chunk_off(wid, k), CHUNK_WORDS)], bufs[s], in_sems[s]
        )

    def wait_in(k):
        s = k % NSLOTS
        pltpu.make_async_copy(
            x_hbm.at[pl.ds(_chunk_off(wid, k), CHUNK_WORDS)], bufs[s], in_sems[s]
        ).wait()

    def start_out(k):
        s = k % NSLOTS
        pltpu.async_copy(
            bufs[s], out_hbm.at[pl.ds(_chunk_off(wid, k), CHUNK_WORDS)], out_sems[s]
        )

    def wait_out(k):
        s = k % NSLOTS
        pltpu.make_async_copy(
            bufs[s], out_hbm.at[pl.ds(_chunk_off(wid, k), CHUNK_WORDS)], out_sems[s]
        ).wait()

    start_in(0)
    start_in(1)

    for k in range(N_CHUNKS):
        s = k % NSLOTS
        wait_in(k)

        pe_off = (k % CHUNKS_PER_BAND) * CHUNK_WORDS
        buf = bufs[s]

        def body(i, _, buf=buf, pe_off=pe_off):
            base = i * (_L * UNROLL)
            for u in range(UNROLL):
                o = base + u * _L
                buf[pl.ds(o, _L)] = buf[pl.ds(o, _L)] + pe_v[pl.ds(pe_off + o, _L)]
            return 0

        lax.fori_loop(0, CHUNK_WORDS // (_L * UNROLL), body, 0)

        start_out(k)
        nxt = k + NSLOTS - 1
        if nxt < N_CHUNKS:
            if nxt >= NSLOTS:
                wait_out(nxt - NSLOTS)
            start_in(nxt)

    for k in range(N_CHUNKS - NSLOTS, N_CHUNKS):
        wait_out(k)


def kernel(x, pe):
    out_flat = _pe_add(x.reshape(-1), pe.reshape(-1))
    return out_flat.reshape(x.shape)


# speedup vs baseline: 1.9958x; 1.6544x over previous
import functools

import jax
import jax.numpy as jnp
from jax import lax
from jax.experimental import pallas as pl
from jax.experimental.pallas import tpu as pltpu
from jax.experimental.pallas import tpu_sc as plsc

D_MODEL = 1024
SEQ_LEN = 2048
BATCH = 4

_INFO = plsc.get_sparse_core_info()
_NC, _NS, _L = _INFO.num_cores, _INFO.num_subcores, _INFO.num_lanes
_NW = _NC * _NS

ROWS_PER_W = SEQ_LEN // _NW
CHUNK_ROWS = 16
CHUNKS_PER_BAND = ROWS_PER_W // CHUNK_ROWS
N_CHUNKS = BATCH * CHUNKS_PER_BAND
NSLOTS = 3
GROUPS_PER_ROW = D_MODEL // _L

_mesh = plsc.VectorSubcoreMesh(core_axis_name="c", subcore_axis_name="s")


@functools.partial(
    pl.kernel,
    mesh=_mesh,
    out_type=jax.ShapeDtypeStruct((BATCH, SEQ_LEN, D_MODEL), jnp.float32),
    scratch_types=[
        pltpu.VMEM((ROWS_PER_W, D_MODEL), jnp.float32),
    ]
    + [pltpu.VMEM((CHUNK_ROWS, D_MODEL), jnp.float32) for _ in range(NSLOTS)]
    + [pltpu.SemaphoreType.DMA for _ in range(2 * NSLOTS)],
)
def _pe_add(x_hbm, pe_hbm, out_hbm, pe_v, b0, b1, b2, i0, i1, i2, o0, o1, o2):
    bufs = (b0, b1, b2)
    in_sems = (i0, i1, i2)
    out_sems = (o0, o1, o2)

    wid = lax.axis_index("s") * _NC + lax.axis_index("c")
    row_base = wid * ROWS_PER_W

    pltpu.sync_copy(pe_hbm.at[pl.ds(row_base, ROWS_PER_W)], pe_v)

    def chunk_slice(k):
        b, c = divmod(k, CHUNKS_PER_BAND)
        return b, row_base + c * CHUNK_ROWS

    def start_in(k):
        s = k % NSLOTS
        b, r0 = chunk_slice(k)
        pltpu.async_copy(x_hbm.at[b, pl.ds(r0, CHUNK_ROWS)], bufs[s], in_sems[s])

    def wait_in(k):
        s = k % NSLOTS
        b, r0 = chunk_slice(k)
        pltpu.make_async_copy(
            x_hbm.at[b, pl.ds(r0, CHUNK_ROWS)], bufs[s], in_sems[s]
        ).wait()

    def start_out(k):
        s = k % NSLOTS
        b, r0 = chunk_slice(k)
        pltpu.async_copy(bufs[s], out_hbm.at[b, pl.ds(r0, CHUNK_ROWS)], out_sems[s])

    def wait_out(k):
        s = k % NSLOTS
        b, r0 = chunk_slice(k)
        pltpu.make_async_copy(
            bufs[s], out_hbm.at[b, pl.ds(r0, CHUNK_ROWS)], out_sems[s]
        ).wait()

    start_in(0)
    start_in(1)

    for k in range(N_CHUNKS):
        s = k % NSLOTS
        wait_in(k)

        pe_r0 = (k % CHUNKS_PER_BAND) * CHUNK_ROWS
        buf = bufs[s]

        def body(r, _, buf=buf, pe_r0=pe_r0):
            pe_r = pe_r0 + r
            for g in range(GROUPS_PER_ROW):
                o = g * _L
                buf[r, pl.ds(o, _L)] = (
                    buf[r, pl.ds(o, _L)] + pe_v[pe_r, pl.ds(o, _L)]
                )
            return 0

        lax.fori_loop(0, CHUNK_ROWS, body, 0)

        start_out(k)
        nxt = k + NSLOTS - 1
        if nxt < N_CHUNKS:
            if nxt >= NSLOTS:
                wait_out(nxt - NSLOTS)
            start_in(nxt)

    for k in range(N_CHUNKS - NSLOTS, N_CHUNKS):
        wait_out(k)


def kernel(x, pe):
    return _pe_add(x, pe)


# speedup vs baseline: 2.8041x; 1.4050x over previous
import functools

import jax
import jax.numpy as jnp
from jax import lax
from jax.experimental import pallas as pl
from jax.experimental.pallas import tpu as pltpu
from jax.experimental.pallas import tpu_sc as plsc

D_MODEL = 1024
SEQ_LEN = 2048
BATCH = 4

_INFO = plsc.get_sparse_core_info()
_NC, _NS, _L = _INFO.num_cores, _INFO.num_subcores, _INFO.num_lanes
_NW = _NC * _NS

ROWS_PER_W = SEQ_LEN // _NW
STEP_ROWS = 8
N_STEPS = ROWS_PER_W // STEP_ROWS
NSLOTS = 3
UNROLL = 2
GROUPS = D_MODEL // _L

_mesh = plsc.VectorSubcoreMesh(core_axis_name="c", subcore_axis_name="s")


@functools.partial(
    pl.kernel,
    mesh=_mesh,
    out_type=jax.ShapeDtypeStruct((BATCH, SEQ_LEN, D_MODEL), jnp.float32),
    scratch_types=(
        [pltpu.VMEM((STEP_ROWS, D_MODEL), jnp.float32) for _ in range(NSLOTS * BATCH)]
        + [pltpu.VMEM((STEP_ROWS, D_MODEL), jnp.float32) for _ in range(NSLOTS)]
        + [pltpu.SemaphoreType.DMA for _ in range(2 * NSLOTS)]
    ),
)
def _pe_add(x_hbm, pe_hbm, out_hbm, *scratch):
    xbufs = tuple(
        tuple(scratch[s * BATCH + b] for b in range(BATCH)) for s in range(NSLOTS)
    )
    pebufs = tuple(scratch[NSLOTS * BATCH + s] for s in range(NSLOTS))
    in_sems = tuple(scratch[NSLOTS * (BATCH + 1) + s] for s in range(NSLOTS))
    out_sems = tuple(scratch[NSLOTS * (BATCH + 2) + s] for s in range(NSLOTS))

    wid = lax.axis_index("s") * _NC + lax.axis_index("c")
    band0 = wid * ROWS_PER_W

    def row0(k):
        return band0 + k * STEP_ROWS

    def start_in(k):
        s = k % NSLOTS
        r0 = row0(k)
        pltpu.async_copy(pe_hbm.at[pl.ds(r0, STEP_ROWS)], pebufs[s], in_sems[s])
        for b in range(BATCH):
            pltpu.async_copy(
                x_hbm.at[b, pl.ds(r0, STEP_ROWS)], xbufs[s][b], in_sems[s]
            )

    def wait_in(k):
        s = k % NSLOTS
        r0 = row0(k)
        pltpu.make_async_copy(
            pe_hbm.at[pl.ds(r0, STEP_ROWS)], pebufs[s], in_sems[s]
        ).wait()
        for b in range(BATCH):
            pltpu.make_async_copy(
                x_hbm.at[b, pl.ds(r0, STEP_ROWS)], xbufs[s][b], in_sems[s]
            ).wait()

    def start_out(k):
        s = k % NSLOTS
        r0 = row0(k)
        for b in range(BATCH):
            pltpu.async_copy(
                xbufs[s][b], out_hbm.at[b, pl.ds(r0, STEP_ROWS)], out_sems[s]
            )

    def wait_out(k):
        s = k % NSLOTS
        r0 = row0(k)
        for b in range(BATCH):
            pltpu.make_async_copy(
                xbufs[s][b], out_hbm.at[b, pl.ds(r0, STEP_ROWS)], out_sems[s]
            ).wait()

    start_in(0)
    start_in(1)

    for k in range(N_STEPS):
        s = k % NSLOTS
        wait_in(k)
        xb = xbufs[s]
        peb = pebufs[s]

        for r in range(STEP_ROWS):

            def body(g, _, xb=xb, peb=peb, r=r):
                col = g * (_L * UNROLL)
                for u in range(UNROLL):
                    o = col + u * _L
                    p = peb[r, pl.ds(o, _L)]
                    for b in range(BATCH):
                        xb[b][r, pl.ds(o, _L)] = xb[b][r, pl.ds(o, _L)] + p
                return 0

            lax.fori_loop(0, GROUPS // UNROLL, body, 0)

        start_out(k)
        nxt = k + NSLOTS - 1
        if nxt < N_STEPS:
            if nxt >= NSLOTS:
                wait_out(nxt - NSLOTS)
            start_in(nxt)

    for k in range(max(0, N_STEPS - NSLOTS), N_STEPS):
        wait_out(k)


def kernel(x, pe):
    return _pe_add(x, pe)


# speedup vs baseline: 2.8474x; 1.0155x over previous
import functools

import jax
import jax.numpy as jnp
from jax import lax
from jax.experimental import pallas as pl
from jax.experimental.pallas import tpu as pltpu
from jax.experimental.pallas import tpu_sc as plsc

D_MODEL = 1024
SEQ_LEN = 2048
BATCH = 4

_INFO = plsc.get_sparse_core_info()
_NC, _NS, _L = _INFO.num_cores, _INFO.num_subcores, _INFO.num_lanes
_NW = _NC * _NS

ROWS_PER_W = SEQ_LEN // _NW
STEP_ROWS = 4
N_STEPS = ROWS_PER_W // STEP_ROWS
NSLOTS = 6
UNROLL = 2
GROUPS = D_MODEL // _L

_mesh = plsc.VectorSubcoreMesh(core_axis_name="c", subcore_axis_name="s")


@functools.partial(
    pl.kernel,
    mesh=_mesh,
    out_type=jax.ShapeDtypeStruct((BATCH, SEQ_LEN, D_MODEL), jnp.float32),
    scratch_types=(
        [pltpu.VMEM((STEP_ROWS, D_MODEL), jnp.float32) for _ in range(NSLOTS * BATCH)]
        + [pltpu.VMEM((STEP_ROWS, D_MODEL), jnp.float32) for _ in range(NSLOTS)]
        + [pltpu.SemaphoreType.DMA for _ in range(2 * NSLOTS)]
    ),
)
def _pe_add(x_hbm, pe_hbm, out_hbm, *scratch):
    xbufs = tuple(
        tuple(scratch[s * BATCH + b] for b in range(BATCH)) for s in range(NSLOTS)
    )
    pebufs = tuple(scratch[NSLOTS * BATCH + s] for s in range(NSLOTS))
    in_sems = tuple(scratch[NSLOTS * (BATCH + 1) + s] for s in range(NSLOTS))
    out_sems = tuple(scratch[NSLOTS * (BATCH + 2) + s] for s in range(NSLOTS))

    wid = lax.axis_index("s") * _NC + lax.axis_index("c")
    band0 = wid * ROWS_PER_W

    def row0(k):
        return band0 + k * STEP_ROWS

    def start_in(k):
        s = k % NSLOTS
        r0 = row0(k)
        pltpu.async_copy(pe_hbm.at[pl.ds(r0, STEP_ROWS)], pebufs[s], in_sems[s])
        for b in range(BATCH):
            pltpu.async_copy(
                x_hbm.at[b, pl.ds(r0, STEP_ROWS)], xbufs[s][b], in_sems[s]
            )

    def wait_in(k):
        s = k % NSLOTS
        r0 = row0(k)
        pltpu.make_async_copy(
            pe_hbm.at[pl.ds(r0, STEP_ROWS)], pebufs[s], in_sems[s]
        ).wait()
        for b in range(BATCH):
            pltpu.make_async_copy(
                x_hbm.at[b, pl.ds(r0, STEP_ROWS)], xbufs[s][b], in_sems[s]
            ).wait()

    def start_out(k):
        s = k % NSLOTS
        r0 = row0(k)
        for b in range(BATCH):
            pltpu.async_copy(
                xbufs[s][b], out_hbm.at[b, pl.ds(r0, STEP_ROWS)], out_sems[s]
            )

    def wait_out(k):
        s = k % NSLOTS
        r0 = row0(k)
        for b in range(BATCH):
            pltpu.make_async_copy(
                xbufs[s][b], out_hbm.at[b, pl.ds(r0, STEP_ROWS)], out_sems[s]
            ).wait()

    for k in range(min(N_STEPS, NSLOTS - 1)):
        start_in(k)

    for k in range(N_STEPS):
        s = k % NSLOTS
        wait_in(k)
        xb = xbufs[s]
        peb = pebufs[s]

        for r in range(STEP_ROWS):

            def body(g, _, xb=xb, peb=peb, r=r):
                col = g * (_L * UNROLL)
                for u in range(UNROLL):
                    o = col + u * _L
                    p = peb[r, pl.ds(o, _L)]
                    for b in range(BATCH):
                        xb[b][r, pl.ds(o, _L)] = xb[b][r, pl.ds(o, _L)] + p
                return 0

            lax.fori_loop(0, GROUPS // UNROLL, body, 0)

        start_out(k)
        nxt = k + NSLOTS - 1
        if nxt < N_STEPS:
            if nxt >= NSLOTS:
                wait_out(nxt - NSLOTS)
            start_in(nxt)

    for k in range(max(0, N_STEPS - NSLOTS), N_STEPS):
        wait_out(k)


def kernel(x, pe):
    return _pe_add(x, pe)
